# Initial kernel scaffold; baseline (speedup 1.0000x reference)
import jax
import jax.numpy as jnp
from jax.experimental import pallas as pl


def kernel(x, edge_index, edge_attr, batch, proj_W, proj_b, enn_W1, enn_b1, enn_W2, enn_b2, root_W, conv_b, gru_W_ih, gru_W_hh, gru_b_ih, gru_b_hh, lstm_W_ih, lstm_W_hh, lstm_b_ih, lstm_b_hh, ffn_W1, ffn_b1, ffn_W2, ffn_b2, ffn_W3, ffn_b3):
    raise NotImplementedError("write your pallas kernel here")


# speedup vs baseline: 2.4413x; 2.4413x over previous
import functools

import jax
import jax.numpy as jnp
from jax import lax
from jax.experimental import pallas as pl
from jax.experimental.pallas import tpu as pltpu
from jax.experimental.pallas import tpu_sc as plsc

NN = 10000
EE = 160000
BB = 64
H = 32
K9 = 9
KP = 16
NC, NS, L = 2, 16, 16
NW = NC * NS
CH = 128
NCHUNK = 40
EPAD = NW * NCHUNK * CH
ER = EPAD // CH
NPAD = 10240
RPT = NPAD // NS
UD = K9 * H


def _prologue_body(x_ref, pw_ref, pb_ref, wf_ref, out_ref, uhat_ref):
    o = jnp.maximum(
        jnp.dot(x_ref[...], pw_ref[...], preferred_element_type=jnp.float32)
        + pb_ref[...], 0.0)
    out_ref[...] = o
    uhat_ref[...] = jnp.dot(o, wf_ref[...], preferred_element_type=jnp.float32)


_HE_BLK = 8192


def _he_body(a_ref, w1_ref, b1_ref, he_ref):
    eid = (pl.program_id(0) * _HE_BLK
           + lax.broadcasted_iota(jnp.int32, (_HE_BLK, KP), 0))
    valid = (eid < EE).astype(jnp.float32)
    he = jnp.maximum(a_ref[...] * w1_ref[...] + b1_ref[...], 0.0)
    he_ref[...] = he * valid


def _update_body(aggp_ref, out_ref, rw_ref, cb_ref,
                 wir_ref, wiz_ref, win_ref, whr_ref, whz_ref, whn_ref,
                 bir_ref, biz_ref, bin_ref, bhr_ref, bhz_ref, bhn_ref,
                 wf_ref, newout_ref, newuhat_ref):
    agg = aggp_ref[0, :NN, :] + aggp_ref[1, :NN, :]
    o = out_ref[...]
    conv = agg + jnp.dot(o, rw_ref[...],
                         preferred_element_type=jnp.float32) + cb_ref[...]
    xg = jnp.maximum(conv, 0.0)
    gxr = jnp.dot(xg, wir_ref[...], preferred_element_type=jnp.float32) + bir_ref[...]
    gxz = jnp.dot(xg, wiz_ref[...], preferred_element_type=jnp.float32) + biz_ref[...]
    gxn = jnp.dot(xg, win_ref[...], preferred_element_type=jnp.float32) + bin_ref[...]
    ghr = jnp.dot(o, whr_ref[...], preferred_element_type=jnp.float32) + bhr_ref[...]
    ghz = jnp.dot(o, whz_ref[...], preferred_element_type=jnp.float32) + bhz_ref[...]
    ghn = jnp.dot(o, whn_ref[...], preferred_element_type=jnp.float32) + bhn_ref[...]
    r = jax.nn.sigmoid(gxr + ghr)
    z = jax.nn.sigmoid(gxz + ghz)
    n = jnp.tanh(gxn + r * ghn)
    h = (1.0 - z) * n + z * o
    newout_ref[...] = h
    newuhat_ref[...] = jnp.dot(h, wf_ref[...], preferred_element_type=jnp.float32)


def _readout_body(out_ref, batch_ref,
                  bii_ref, big_ref, bio_ref, bhi_ref, bhg_ref, bho_ref,
                  w1a_ref, w1b_ref, b1_ref, w2_ref, b2_ref, w3_ref, b3_ref,
                  y_ref):
    ig = jax.nn.sigmoid(bii_ref[...] + bhi_ref[...])
    gg = jnp.tanh(big_ref[...] + bhg_ref[...])
    og = jax.nn.sigmoid(bio_ref[...] + bho_ref[...])
    q = og * jnp.tanh(ig * gg)
    o = out_ref[...]
    oh = (batch_ref[...] ==
          lax.broadcasted_iota(jnp.int32, (NN, BB), 1)).astype(jnp.float32)
    e = jnp.sum(o * q, axis=1, keepdims=True)
    masked = jnp.where(oh > 0.0, e, -1e30)
    emax = jnp.max(masked, axis=0, keepdims=True)
    emax_row = jnp.sum(oh * emax, axis=1, keepdims=True)
    ee = jnp.exp(e - emax_row)
    denom = jnp.sum(oh * ee, axis=0, keepdims=True)
    denom_row = jnp.sum(oh * denom, axis=1, keepdims=True)
    a = ee / denom_row
    rvec = lax.dot_general(oh, a * o, (((0,), (0,)), ((), ())),
                           preferred_element_type=jnp.float32)
    y1 = jnp.maximum(
        jnp.dot(q, w1a_ref[...], preferred_element_type=jnp.float32)
        + jnp.dot(rvec, w1b_ref[...], preferred_element_type=jnp.float32)
        + b1_ref[...], 0.0)
    y2 = jnp.maximum(
        jnp.dot(y1, w2_ref[...], preferred_element_type=jnp.float32)
        + b2_ref[...], 0.0)
    y_ref[...] = (jnp.dot(y2, w3_ref[...], preferred_element_type=jnp.float32)
                  + b3_ref[...])


_prologue = pl.pallas_call(
    _prologue_body,
    out_shape=[
        jax.ShapeDtypeStruct((NN, H), jnp.float32),
        jax.ShapeDtypeStruct((NN, UD), jnp.float32),
    ],
)

_he_calc = pl.pallas_call(
    _he_body,
    grid=(EPAD // _HE_BLK,),
    in_specs=[
        pl.BlockSpec((_HE_BLK, 1), lambda i: (i, 0)),
        pl.BlockSpec((1, KP), lambda i: (0, 0)),
        pl.BlockSpec((1, KP), lambda i: (0, 0)),
    ],
    out_specs=pl.BlockSpec((_HE_BLK, KP), lambda i: (i, 0)),
    out_shape=jax.ShapeDtypeStruct((EPAD, KP), jnp.float32),
)

_update = pl.pallas_call(
    _update_body,
    out_shape=[
        jax.ShapeDtypeStruct((NN, H), jnp.float32),
        jax.ShapeDtypeStruct((NN, UD), jnp.float32),
    ],
)

_readout = pl.pallas_call(
    _readout_body,
    out_shape=jax.ShapeDtypeStruct((BB, 1), jnp.float32),
)


def _sc_body(uhat_hbm, src_hbm, dst_hbm, he_hbm, out_hbm,
             src_v, dst_v, he_v, rows_v, msg_v, agg_sh, sem):
    cid = lax.axis_index("c")
    sid = lax.axis_index("s")
    wid = sid * NC + cid

    def zrow(i, carry):
        msg_v[i, pl.ds(0, L)] = jnp.zeros((L,), jnp.float32)
        msg_v[i, pl.ds(L, L)] = jnp.zeros((L,), jnp.float32)
        return carry
    lax.fori_loop(0, CH, zrow, 0)
    for b in range(RPT // CH):
        pltpu.sync_copy(msg_v, agg_sh.at[pl.ds(sid * RPT + b * CH, CH)])
    plsc.subcore_barrier()

    def chunk(g, carry):
        base = (wid * NCHUNK + g) * CH
        pltpu.sync_copy(src_hbm.at[pl.ds(base, CH)], src_v)
        pltpu.sync_copy(dst_hbm.at[pl.ds(base, CH)], dst_v)
        pltpu.sync_copy(he_hbm.at[pl.ds(base, CH)], he_v)
        pltpu.async_copy(uhat_hbm.at[src_v], rows_v, sem).wait()

        def edge(c, ecarry):
            hv = he_v[c, :]
            m0 = jnp.zeros((L,), jnp.float32)
            m1 = jnp.zeros((L,), jnp.float32)
            for k in range(K9):
                hk = hv[k]
                m0 = m0 + hk * rows_v[c, pl.ds(k * H, L)]
                m1 = m1 + hk * rows_v[c, pl.ds(k * H + L, L)]
            msg_v[c, pl.ds(0, L)] = m0
            msg_v[c, pl.ds(L, L)] = m1
            return ecarry
        lax.fori_loop(0, CH, edge, 0)
        pltpu.sync_copy(msg_v, agg_sh.at[dst_v], add=True)
        return carry
    lax.fori_loop(0, NCHUNK, chunk, 0)
    plsc.subcore_barrier()

    for b in range(RPT // CH):
        off = sid * RPT + b * CH
        pltpu.sync_copy(agg_sh.at[pl.ds(off, CH)], msg_v)
        pltpu.sync_copy(msg_v, out_hbm.at[cid, pl.ds(off, CH)])


_sc_step = pl.kernel(
    _sc_body,
    out_type=jax.ShapeDtypeStruct((NC, NPAD, H), jnp.float32),
    mesh=plsc.VectorSubcoreMesh(core_axis_name="c", subcore_axis_name="s",
                                num_cores=NC, num_subcores=NS),
    compiler_params=pltpu.CompilerParams(use_tc_tiling_on_sc=False),
    scratch_types=[
        pltpu.VMEM((CH,), jnp.int32),
        pltpu.VMEM((CH,), jnp.int32),
        pltpu.VMEM((CH, KP), jnp.float32),
        pltpu.VMEM((CH, UD), jnp.float32),
        pltpu.VMEM((CH, H), jnp.float32),
        pltpu.VMEM_SHARED((NPAD, H), jnp.float32),
        pltpu.SemaphoreType.DMA,
    ],
)


def kernel(x, edge_index, edge_attr, batch,
           proj_W, proj_b, enn_W1, enn_b1, enn_W2, enn_b2, root_W, conv_b,
           gru_W_ih, gru_W_hh, gru_b_ih, gru_b_hh,
           lstm_W_ih, lstm_W_hh, lstm_b_ih, lstm_b_hh,
           ffn_W1, ffn_b1, ffn_W2, ffn_b2, ffn_W3, ffn_b3):
    f32 = jnp.float32
    src = edge_index[0].astype(jnp.int32)
    dst = edge_index[1].astype(jnp.int32)
    pad_e = EPAD - EE
    src_p = jnp.concatenate([src, jnp.zeros((pad_e,), jnp.int32)])
    dst_p = jnp.concatenate([dst, jnp.zeros((pad_e,), jnp.int32)])
    a_col = jnp.concatenate([edge_attr.astype(f32),
                             jnp.zeros((pad_e,), f32)]).reshape(EPAD, 1)

    w1row = jnp.concatenate([enn_W1[:, 0],
                             jnp.zeros((KP - 8,), f32)]).reshape(1, KP)
    b1row = jnp.concatenate([enn_b1, jnp.ones((1,), f32),
                             jnp.zeros((KP - K9,), f32)]).reshape(1, KP)

    w2aug = jnp.concatenate([enn_W2, enn_b2[:, None]], axis=1)
    wfold = w2aug.reshape(H, H, K9).transpose(0, 2, 1).reshape(H, UD)

    projWT = proj_W.T
    projb = proj_b.reshape(1, H)
    rootWT = root_W.T
    convb = conv_b.reshape(1, H)
    wir, wiz, win = (gru_W_ih[0:H].T, gru_W_ih[H:2 * H].T, gru_W_ih[2 * H:].T)
    whr, whz, whn = (gru_W_hh[0:H].T, gru_W_hh[H:2 * H].T, gru_W_hh[2 * H:].T)
    bir, biz, bin_ = (gru_b_ih[0:H].reshape(1, H),
                      gru_b_ih[H:2 * H].reshape(1, H),
                      gru_b_ih[2 * H:].reshape(1, H))
    bhr, bhz, bhn = (gru_b_hh[0:H].reshape(1, H),
                     gru_b_hh[H:2 * H].reshape(1, H),
                     gru_b_hh[2 * H:].reshape(1, H))
    bii, big, bio = (lstm_b_ih[0:H].reshape(1, H),
                     lstm_b_ih[2 * H:3 * H].reshape(1, H),
                     lstm_b_ih[3 * H:].reshape(1, H))
    bhi, bhg, bho = (lstm_b_hh[0:H].reshape(1, H),
                     lstm_b_hh[2 * H:3 * H].reshape(1, H),
                     lstm_b_hh[3 * H:].reshape(1, H))
    w1a = ffn_W1[:, 0:H].T
    w1b_f = ffn_W1[:, H:].T
    fb1 = ffn_b1.reshape(1, H)
    w2T = ffn_W2.T
    fb2 = ffn_b2.reshape(1, H)
    w3T = ffn_W3.T
    fb3 = ffn_b3.reshape(1, 1)
    batch2d = batch.astype(jnp.int32).reshape(NN, 1)

    out_t, uhat_t = _prologue(x, projWT, projb, wfold)
    he9 = _he_calc(a_col, w1row, b1row)
    for _ in range(3):
        aggp = _sc_step(uhat_t, src_p, dst_p, he9)
        out_t, uhat_t = _update(aggp, out_t, rootWT, convb,
                                wir, wiz, win, whr, whz, whn,
                                bir, biz, bin_, bhr, bhz, bhn, wfold)
    y = _readout(out_t, batch2d, bii, big, bio, bhi, bhg, bho,
                 w1a, w1b_f, fb1, w2T, fb2, w3T, fb3)
    return y


# speedup vs baseline: 3.4514x; 1.4138x over previous
import functools

import jax
import jax.numpy as jnp
from jax import lax
from jax.experimental import pallas as pl
from jax.experimental.pallas import tpu as pltpu
from jax.experimental.pallas import tpu_sc as plsc

NN = 10000
EE = 160000
BB = 64
H = 32
K9 = 9
KP = 16
NC, NS, L = 2, 16, 16
NW = NC * NS
CH = 128
NCHUNK = 40
EPAD = NW * NCHUNK * CH
ER = EPAD // CH
NPAD = 10240
RPT = NPAD // NS
UD = K9 * H


def _prologue_body(x_ref, pw_ref, pb_ref, wf_ref, out_ref, uhat_ref):
    o = jnp.maximum(
        jnp.dot(x_ref[...], pw_ref[...], preferred_element_type=jnp.float32)
        + pb_ref[...], 0.0)
    out_ref[...] = o
    uhat_ref[...] = jnp.dot(o, wf_ref[...], preferred_element_type=jnp.float32)


_HE_BLK = 8192


def _he_body(a_ref, w1_ref, b1_ref, he_ref):
    eid = (pl.program_id(0) * _HE_BLK
           + lax.broadcasted_iota(jnp.int32, (_HE_BLK, KP), 0))
    valid = (eid < EE).astype(jnp.float32)
    he = jnp.maximum(a_ref[...] * w1_ref[...] + b1_ref[...], 0.0)
    he_ref[...] = (he * valid).astype(jnp.bfloat16).astype(jnp.float32)


def _update_body(aggp_ref, out_ref, rw_ref, cb_ref,
                 wir_ref, wiz_ref, win_ref, whr_ref, whz_ref, whn_ref,
                 bir_ref, biz_ref, bin_ref, bhr_ref, bhz_ref, bhn_ref,
                 wf_ref, newout_ref, newuhat_ref):
    agg = aggp_ref[0] + aggp_ref[1]
    o = out_ref[...]
    conv = agg + jnp.dot(o, rw_ref[...],
                         preferred_element_type=jnp.float32) + cb_ref[...]
    xg = jnp.maximum(conv, 0.0)
    gxr = jnp.dot(xg, wir_ref[...], preferred_element_type=jnp.float32) + bir_ref[...]
    gxz = jnp.dot(xg, wiz_ref[...], preferred_element_type=jnp.float32) + biz_ref[...]
    gxn = jnp.dot(xg, win_ref[...], preferred_element_type=jnp.float32) + bin_ref[...]
    ghr = jnp.dot(o, whr_ref[...], preferred_element_type=jnp.float32) + bhr_ref[...]
    ghz = jnp.dot(o, whz_ref[...], preferred_element_type=jnp.float32) + bhz_ref[...]
    ghn = jnp.dot(o, whn_ref[...], preferred_element_type=jnp.float32) + bhn_ref[...]
    r = jax.nn.sigmoid(gxr + ghr)
    z = jax.nn.sigmoid(gxz + ghz)
    n = jnp.tanh(gxn + r * ghn)
    h = (1.0 - z) * n + z * o
    newout_ref[...] = h
    newuhat_ref[...] = jnp.dot(h, wf_ref[...], preferred_element_type=jnp.float32)


def _readout_body(out_ref, batch_ref,
                  bii_ref, big_ref, bio_ref, bhi_ref, bhg_ref, bho_ref,
                  w1a_ref, w1b_ref, b1_ref, w2_ref, b2_ref, w3_ref, b3_ref,
                  y_ref):
    ig = jax.nn.sigmoid(bii_ref[...] + bhi_ref[...])
    gg = jnp.tanh(big_ref[...] + bhg_ref[...])
    og = jax.nn.sigmoid(bio_ref[...] + bho_ref[...])
    q = og * jnp.tanh(ig * gg)
    o = out_ref[...]
    oh = (batch_ref[...] ==
          lax.broadcasted_iota(jnp.int32, (NN, BB), 1)).astype(jnp.float32)
    e = jnp.sum(o * q, axis=1, keepdims=True)
    masked = jnp.where(oh > 0.0, e, -1e30)
    emax = jnp.max(masked, axis=0, keepdims=True)
    emax_row = jnp.sum(oh * emax, axis=1, keepdims=True)
    ee = jnp.exp(e - emax_row)
    denom = jnp.sum(oh * ee, axis=0, keepdims=True)
    denom_row = jnp.sum(oh * denom, axis=1, keepdims=True)
    a = ee / denom_row
    rvec = lax.dot_general(oh, a * o, (((0,), (0,)), ((), ())),
                           preferred_element_type=jnp.float32,
                           precision=lax.Precision.HIGHEST)
    y1 = jnp.maximum(
        jnp.dot(q, w1a_ref[...], preferred_element_type=jnp.float32)
        + jnp.dot(rvec, w1b_ref[...], preferred_element_type=jnp.float32)
        + b1_ref[...], 0.0)
    y2 = jnp.maximum(
        jnp.dot(y1, w2_ref[...], preferred_element_type=jnp.float32)
        + b2_ref[...], 0.0)
    y_ref[...] = (jnp.dot(y2, w3_ref[...], preferred_element_type=jnp.float32)
                  + b3_ref[...])


_NB = 2000

_prologue = pl.pallas_call(
    _prologue_body,
    grid=(NN // _NB,),
    in_specs=[
        pl.BlockSpec((_NB, 4), lambda i: (i, 0)),
        pl.BlockSpec((4, H), lambda i: (0, 0)),
        pl.BlockSpec((1, H), lambda i: (0, 0)),
        pl.BlockSpec((H, UD), lambda i: (0, 0)),
    ],
    out_specs=[
        pl.BlockSpec((_NB, H), lambda i: (i, 0)),
        pl.BlockSpec((_NB, UD), lambda i: (i, 0)),
    ],
    out_shape=[
        jax.ShapeDtypeStruct((NN, H), jnp.float32),
        jax.ShapeDtypeStruct((NN, UD), jnp.float32),
    ],
)

_he_calc = pl.pallas_call(
    _he_body,
    grid=(EPAD // _HE_BLK,),
    in_specs=[
        pl.BlockSpec((_HE_BLK, 1), lambda i: (i, 0)),
        pl.BlockSpec((1, KP), lambda i: (0, 0)),
        pl.BlockSpec((1, KP), lambda i: (0, 0)),
    ],
    out_specs=pl.BlockSpec((_HE_BLK, KP), lambda i: (i, 0)),
    out_shape=jax.ShapeDtypeStruct((EPAD, KP), jnp.float32),
)

_update = pl.pallas_call(
    _update_body,
    grid=(NN // _NB,),
    in_specs=([pl.BlockSpec((NC, _NB, H), lambda i: (0, i, 0)),
               pl.BlockSpec((_NB, H), lambda i: (i, 0)),
               pl.BlockSpec((H, H), lambda i: (0, 0)),
               pl.BlockSpec((1, H), lambda i: (0, 0))]
              + [pl.BlockSpec((H, H), lambda i: (0, 0)) for _ in range(6)]
              + [pl.BlockSpec((1, H), lambda i: (0, 0)) for _ in range(6)]
              + [pl.BlockSpec((H, UD), lambda i: (0, 0))]),
    out_specs=[
        pl.BlockSpec((_NB, H), lambda i: (i, 0)),
        pl.BlockSpec((_NB, UD), lambda i: (i, 0)),
    ],
    out_shape=[
        jax.ShapeDtypeStruct((NN, H), jnp.float32),
        jax.ShapeDtypeStruct((NN, UD), jnp.float32),
    ],
)

_readout = pl.pallas_call(
    _readout_body,
    out_shape=jax.ShapeDtypeStruct((BB, 1), jnp.float32),
)


_UNROLL = 4


def _sc_body(uhat_hbm, src_hbm, dst_hbm, he_hbm, out_hbm,
             srcs_v, dsts_v, he0_v, he1_v, rows0_v, rows1_v,
             msg0_v, msg1_v, agg_sh, sg0, sg1, sh0, sh1, ss0, ss1):
    cid = lax.axis_index("c")
    sid = lax.axis_index("s")
    wid = sid * NC + cid
    hes = (he0_v, he1_v)
    rows = (rows0_v, rows1_v)
    msgs = (msg0_v, msg1_v)
    sgs = (sg0, sg1)
    shs = (sh0, sh1)
    sss = (ss0, ss1)

    pltpu.sync_copy(src_hbm.at[pl.ds(wid * NCHUNK, NCHUNK)], srcs_v)
    pltpu.sync_copy(dst_hbm.at[pl.ds(wid * NCHUNK, NCHUNK)], dsts_v)

    def zrow(i, carry):
        msg0_v[i, pl.ds(0, L)] = jnp.zeros((L,), jnp.float32)
        msg0_v[i, pl.ds(L, L)] = jnp.zeros((L,), jnp.float32)
        return carry
    lax.fori_loop(0, CH, zrow, 0)
    for b in range(RPT // CH):
        pltpu.sync_copy(msg0_v, agg_sh.at[pl.ds(sid * RPT + b * CH, CH)])
    plsc.subcore_barrier()

    def issue(g, b):
        ebase = (wid * NCHUNK + g) * CH
        pltpu.async_copy(he_hbm.at[pl.ds(ebase, CH)], hes[b], shs[b])
        pltpu.async_copy(uhat_hbm.at[srcs_v.at[g]], rows[b], sgs[b])

    issue(0, 0)
    issue(1, 1)

    def pair(i, carry):
        for b in range(2):
            g = i * 2 + b
            ebase = (wid * NCHUNK + g) * CH
            pltpu.make_async_copy(he_hbm.at[pl.ds(ebase, CH)], hes[b],
                                  shs[b]).wait()
            pltpu.make_async_copy(uhat_hbm.at[srcs_v.at[g]], rows[b],
                                  sgs[b]).wait()

            @pl.when(g >= 2)
            def _():
                pltpu.make_async_copy(msgs[b], agg_sh.at[dsts_v.at[g - 2]],
                                      sss[b]).wait()

            def edgeu(j, ecarry):
                for u in range(_UNROLL):
                    c = j * _UNROLL + u
                    hv = hes[b][c, :]
                    m0 = hv[0] * rows[b][c, pl.ds(0, L)]
                    m1 = hv[0] * rows[b][c, pl.ds(L, L)]
                    for k in range(1, K9):
                        hk = hv[k]
                        m0 = m0 + hk * rows[b][c, pl.ds(k * H, L)]
                        m1 = m1 + hk * rows[b][c, pl.ds(k * H + L, L)]
                    msgs[b][c, pl.ds(0, L)] = m0
                    msgs[b][c, pl.ds(L, L)] = m1
                return ecarry
            lax.fori_loop(0, CH // _UNROLL, edgeu, 0)

            pltpu.async_copy(msgs[b], agg_sh.at[dsts_v.at[g]], sss[b],
                             add=True)

            @pl.when(g + 2 < NCHUNK)
            def _():
                issue(g + 2, b)
        return carry
    lax.fori_loop(0, NCHUNK // 2, pair, 0)

    for b in range(2):
        g = NCHUNK - 2 + b
        pltpu.make_async_copy(msgs[b], agg_sh.at[dsts_v.at[g]],
                              sss[b]).wait()
    plsc.subcore_barrier()

    for b in range(RPT // CH):
        off = sid * RPT + b * CH
        pltpu.sync_copy(agg_sh.at[pl.ds(off, CH)], msg0_v)
        pltpu.sync_copy(msg0_v, out_hbm.at[cid, pl.ds(off, CH)])


_sc_step = pl.kernel(
    _sc_body,
    out_type=jax.ShapeDtypeStruct((NC, NPAD, H), jnp.float32),
    mesh=plsc.VectorSubcoreMesh(core_axis_name="c", subcore_axis_name="s",
                                num_cores=NC, num_subcores=NS),
    compiler_params=pltpu.CompilerParams(use_tc_tiling_on_sc=False),
    scratch_types=[
        pltpu.VMEM((NCHUNK, CH), jnp.int32),
        pltpu.VMEM((NCHUNK, CH), jnp.int32),
        pltpu.VMEM((CH, KP), jnp.float32),
        pltpu.VMEM((CH, KP), jnp.float32),
        pltpu.VMEM((CH, UD), jnp.float32),
        pltpu.VMEM((CH, UD), jnp.float32),
        pltpu.VMEM((CH, H), jnp.float32),
        pltpu.VMEM((CH, H), jnp.float32),
        pltpu.VMEM_SHARED((NPAD, H), jnp.float32),
        pltpu.SemaphoreType.DMA,
        pltpu.SemaphoreType.DMA,
        pltpu.SemaphoreType.DMA,
        pltpu.SemaphoreType.DMA,
        pltpu.SemaphoreType.DMA,
        pltpu.SemaphoreType.DMA,
    ],
)


def kernel(x, edge_index, edge_attr, batch,
           proj_W, proj_b, enn_W1, enn_b1, enn_W2, enn_b2, root_W, conv_b,
           gru_W_ih, gru_W_hh, gru_b_ih, gru_b_hh,
           lstm_W_ih, lstm_W_hh, lstm_b_ih, lstm_b_hh,
           ffn_W1, ffn_b1, ffn_W2, ffn_b2, ffn_W3, ffn_b3):
    f32 = jnp.float32
    src = edge_index[0].astype(jnp.int32)
    dst = edge_index[1].astype(jnp.int32)
    pad_e = EPAD - EE
    src_p = jnp.concatenate([src, jnp.zeros((pad_e,), jnp.int32)]).reshape(ER, CH)
    dst_p = jnp.concatenate([dst, jnp.zeros((pad_e,), jnp.int32)]).reshape(ER, CH)
    a_col = jnp.concatenate([edge_attr.astype(f32),
                             jnp.zeros((pad_e,), f32)]).reshape(EPAD, 1)

    w1row = jnp.concatenate([enn_W1[:, 0],
                             jnp.zeros((KP - 8,), f32)]).reshape(1, KP)
    b1row = jnp.concatenate([enn_b1, jnp.ones((1,), f32),
                             jnp.zeros((KP - K9,), f32)]).reshape(1, KP)

    w2aug = jnp.concatenate([enn_W2, enn_b2[:, None]], axis=1)
    wfold = w2aug.reshape(H, H, K9).transpose(0, 2, 1).reshape(H, UD)

    projWT = proj_W.T
    projb = proj_b.reshape(1, H)
    rootWT = root_W.T
    convb = conv_b.reshape(1, H)
    wir, wiz, win = (gru_W_ih[0:H].T, gru_W_ih[H:2 * H].T, gru_W_ih[2 * H:].T)
    whr, whz, whn = (gru_W_hh[0:H].T, gru_W_hh[H:2 * H].T, gru_W_hh[2 * H:].T)
    bir, biz, bin_ = (gru_b_ih[0:H].reshape(1, H),
                      gru_b_ih[H:2 * H].reshape(1, H),
                      gru_b_ih[2 * H:].reshape(1, H))
    bhr, bhz, bhn = (gru_b_hh[0:H].reshape(1, H),
                     gru_b_hh[H:2 * H].reshape(1, H),
                     gru_b_hh[2 * H:].reshape(1, H))
    bii, big, bio = (lstm_b_ih[0:H].reshape(1, H),
                     lstm_b_ih[2 * H:3 * H].reshape(1, H),
                     lstm_b_ih[3 * H:].reshape(1, H))
    bhi, bhg, bho = (lstm_b_hh[0:H].reshape(1, H),
                     lstm_b_hh[2 * H:3 * H].reshape(1, H),
                     lstm_b_hh[3 * H:].reshape(1, H))
    w1a = ffn_W1[:, 0:H].T
    w1b_f = ffn_W1[:, H:].T
    fb1 = ffn_b1.reshape(1, H)
    w2T = ffn_W2.T
    fb2 = ffn_b2.reshape(1, H)
    w3T = ffn_W3.T
    fb3 = ffn_b3.reshape(1, 1)
    batch2d = batch.astype(jnp.int32).reshape(NN, 1)

    out_t, uhat_t = _prologue(x, projWT, projb, wfold)
    he9 = _he_calc(a_col, w1row, b1row)
    for _ in range(3):
        aggp = _sc_step(uhat_t, src_p, dst_p, he9)[:, :NN, :]
        out_t, uhat_t = _update(aggp, out_t, rootWT, convb,
                                wir, wiz, win, whr, whz, whn,
                                bir, biz, bin_, bhr, bhz, bhn, wfold)
    y = _readout(out_t, batch2d, bii, big, bio, bhi, bhg, bho,
                 w1a, w1b_f, fb1, w2T, fb2, w3T, fb3)
    return y
